# baseline (speedup 1.0000x reference)
import functools

import jax
import jax.numpy as jnp
from jax import lax
from jax.experimental import pallas as pl
from jax.experimental.pallas import tpu as pltpu
from jax.experimental.pallas import tpu_sc as plsc

NUM_FIELDS = 26
VOCAB = 100000
EMB = 16
BATCH = 16384
CONT = 13
OUT_W = CONT + NUM_FIELDS * EMB

VPAD = 100096
NTILE = 781
TAIL_V = NTILE * 128
TAIL_N = VOCAB - TAIL_V

_info = plsc.get_sparse_core_info()
NC, NS = _info.num_cores, _info.num_subcores
NW = NC * NS
BPW = BATCH // NW
CHUNK = 128
NCHUNK = BPW // CHUNK

_mesh = plsc.VectorSubcoreMesh(core_axis_name="c", subcore_axis_name="s")


@functools.partial(
    pl.kernel,
    out_type=jax.ShapeDtypeStruct((NUM_FIELDS, VPAD * EMB // 128, 128), jnp.float32),
    mesh=_mesh,
    compiler_params=pltpu.CompilerParams(
        use_tc_tiling_on_sc=True, needs_layout_passes=False
    ),
    scratch_types=[
        pltpu.VMEM((EMB, 128), jnp.float32),
        pltpu.VMEM((16, 128), jnp.float32),
        pltpu.VMEM((4, 128), jnp.float32),
    ],
)
def _sc_format(tq_hbm, tail_hbm, tr_hbm, tilebuf, rowbuf, tailbuf):
    wid = lax.axis_index("s") * NC + lax.axis_index("c")
    base = wid * 24 + jnp.minimum(wid, 13)
    nj = jnp.where(wid < 13, 25, 24)
    iota = lax.iota(jnp.int32, 16)

    for f in range(NUM_FIELDS):

        def jbody(k, carry, f=f):
            j = base + k
            pltpu.sync_copy(tq_hbm.at[f, :, pl.ds(j * 128, 128)], tilebuf)

            def cblk(c8, carry2):
                for u in range(16):
                    col = c8 * 16 + u
                    vals = plsc.load_gather(
                        tilebuf, [iota, jnp.full((16,), 0, jnp.int32) + col]
                    )
                    rowbuf[2 * c8 + u // 8, pl.ds((u % 8) * 16, 16)] = vals
                return carry2

            lax.fori_loop(0, 8, cblk, 0)
            pltpu.sync_copy(rowbuf, tr_hbm.at[f, pl.ds(j * 16, 16)])
            return carry

        lax.fori_loop(0, nj, jbody, 0)

    @pl.when(wid < NUM_FIELDS)
    def _():
        pltpu.sync_copy(tail_hbm.at[wid], tailbuf)
        pltpu.sync_copy(tailbuf, tr_hbm.at[wid, pl.ds(TAIL_V * EMB // 128, 4)])


@functools.partial(
    pl.kernel,
    out_type=jax.ShapeDtypeStruct((BATCH, OUT_W), jnp.float32),
    mesh=_mesh,
    compiler_params=pltpu.CompilerParams(use_tc_tiling_on_sc=False),
    scratch_types=[
        pltpu.VMEM((NUM_FIELDS, CHUNK), jnp.int32),
        pltpu.VMEM((NUM_FIELDS * CHUNK, EMB), jnp.float32),
        pltpu.VMEM((CHUNK * CONT + 16,), jnp.float32),
        pltpu.VMEM((CHUNK, OUT_W), jnp.float32),
        pltpu.SemaphoreType.DMA,
    ],
)
def _sc_embed(xcatT_hbm, xcf_hbm, tables_hbm, out_hbm, idx_v, fgath, cont_v, outbuf, sem):
    wid = lax.axis_index("s") * NC + lax.axis_index("c")

    def chunk_body(c, carry):
        b0 = wid * BPW + c * CHUNK
        pltpu.sync_copy(xcatT_hbm.at[:, pl.ds(b0, CHUNK)], idx_v)
        pltpu.sync_copy(
            xcf_hbm.at[pl.ds(b0 * CONT, CHUNK * CONT)],
            cont_v.at[pl.ds(0, CHUNK * CONT)],
        )
        copies = [
            pltpu.async_copy(
                tables_hbm.at[f].at[idx_v.at[f]],
                fgath.at[pl.ds(f * CHUNK, CHUNK)],
                sem,
            )
            for f in range(NUM_FIELDS)
        ]
        for cp in copies:
            cp.wait()

        def row_body(i, carry2):
            outbuf[i, pl.ds(0, 16)] = cont_v[pl.ds(i * CONT, 16)]
            for f in range(NUM_FIELDS):
                outbuf[i, pl.ds(CONT + f * EMB, EMB)] = fgath[f * CHUNK + i, :]
            return carry2

        lax.fori_loop(0, CHUNK, row_body, 0)
        pltpu.sync_copy(outbuf, out_hbm.at[pl.ds(b0, CHUNK)])
        return carry

    lax.fori_loop(0, NCHUNK, chunk_body, 0)


def kernel(x_cont, x_cat, tables):
    tq = tables.transpose(0, 2, 1)
    tail = tables[:, TAIL_V:, :].reshape(NUM_FIELDS, 4, 128)
    tr = _sc_format(tq, tail)
    xcatT = x_cat.T
    xcf = x_cont.reshape(-1)
    return _sc_embed(xcatT, xcf, tr.reshape(NUM_FIELDS, VPAD, EMB))


# speedup vs baseline: 1.3337x; 1.3337x over previous
import functools

import jax
import jax.numpy as jnp
from jax import lax
from jax.experimental import pallas as pl
from jax.experimental.pallas import tpu as pltpu
from jax.experimental.pallas import tpu_sc as plsc

NUM_FIELDS = 26
VOCAB = 100000
EMB = 16
BATCH = 16384
CONT = 13
OUT_W = CONT + NUM_FIELDS * EMB

VPAD = 100096
NTILE = 781
TAIL_V = NTILE * 128
TAIL_N = VOCAB - TAIL_V

_info = plsc.get_sparse_core_info()
NC, NS = _info.num_cores, _info.num_subcores
NW = NC * NS
BPW = BATCH // NW
CHUNK = 128
NCHUNK = BPW // CHUNK

_mesh = plsc.VectorSubcoreMesh(core_axis_name="c", subcore_axis_name="s")


@functools.partial(
    pl.kernel,
    out_type=jax.ShapeDtypeStruct((NUM_FIELDS, VPAD * EMB // 128, 128), jnp.float32),
    mesh=_mesh,
    compiler_params=pltpu.CompilerParams(
        use_tc_tiling_on_sc=True, needs_layout_passes=False
    ),
    scratch_types=[
        pltpu.VMEM((EMB, 24 * 128), jnp.float32),
        pltpu.VMEM((24 * EMB, 128), jnp.float32),
        pltpu.VMEM((EMB, 128), jnp.float32),
        pltpu.VMEM((16, 128), jnp.float32),
        pltpu.VMEM((4, 128), jnp.float32),
        pltpu.SemaphoreType.DMA,
        pltpu.SemaphoreType.DMA,
    ],
)
def _sc_format(tq_hbm, tail_hbm, tr_hbm, tilebuf, rowbuf, tb1, rb1, tailbuf, isem, osem):
    wid = lax.axis_index("s") * NC + lax.axis_index("c")
    base = wid * 24
    iota = lax.iota(jnp.int32, 16)

    def in_copy(f):
        return pltpu.make_async_copy(
            tq_hbm.at[f, :, pl.ds(base * 128, 24 * 128)], tilebuf, isem
        )

    def out_copy(f):
        return pltpu.make_async_copy(
            rowbuf, tr_hbm.at[f, pl.ds(base * 16, 24 * 16)], osem
        )

    in_copy(0).start()

    def fbody(f, carry):
        in_copy(f).wait()

        @pl.when(f > 0)
        def _():
            out_copy(f - 1).wait()

        def tblock(t, carry2):
            def cblk(c8, carry3):
                for u in range(16):
                    col = t * 128 + c8 * 16 + u
                    vals = plsc.load_gather(
                        tilebuf, [iota, jnp.full((16,), 0, jnp.int32) + col]
                    )
                    rowbuf[t * 16 + 2 * c8 + u // 8, pl.ds((u % 8) * 16, 16)] = vals
                return carry3

            lax.fori_loop(0, 8, cblk, 0)
            return carry2

        lax.fori_loop(0, 24, tblock, 0)
        out_copy(f).start()

        @pl.when(f < NUM_FIELDS - 1)
        def _():
            in_copy(f + 1).start()

        return carry

    lax.fori_loop(0, NUM_FIELDS, fbody, 0)
    out_copy(NUM_FIELDS - 1).wait()

    @pl.when(wid < 13)
    def _():
        j = 768 + wid

        def lbody(f, carry):
            pltpu.sync_copy(tq_hbm.at[f, :, pl.ds(j * 128, 128)], tb1)

            def cblk(c8, carry2):
                for u in range(16):
                    col = c8 * 16 + u
                    vals = plsc.load_gather(
                        tb1, [iota, jnp.full((16,), 0, jnp.int32) + col]
                    )
                    rb1[2 * c8 + u // 8, pl.ds((u % 8) * 16, 16)] = vals
                return carry2

            lax.fori_loop(0, 8, cblk, 0)
            pltpu.sync_copy(rb1, tr_hbm.at[f, pl.ds(j * 16, 16)])
            return carry

        lax.fori_loop(0, NUM_FIELDS, lbody, 0)

    @pl.when(wid < NUM_FIELDS)
    def _():
        pltpu.sync_copy(tail_hbm.at[wid], tailbuf)
        pltpu.sync_copy(tailbuf, tr_hbm.at[wid, pl.ds(TAIL_V * EMB // 128, 4)])


@functools.partial(
    pl.kernel,
    out_type=jax.ShapeDtypeStruct((BATCH, OUT_W), jnp.float32),
    mesh=_mesh,
    compiler_params=pltpu.CompilerParams(use_tc_tiling_on_sc=False),
    scratch_types=[
        pltpu.VMEM((NUM_FIELDS, CHUNK), jnp.int32),
        pltpu.VMEM((NUM_FIELDS * CHUNK, EMB), jnp.float32),
        pltpu.VMEM((CHUNK * CONT + 16,), jnp.float32),
        pltpu.VMEM((CHUNK, OUT_W), jnp.float32),
        pltpu.SemaphoreType.DMA,
    ],
)
def _sc_embed(xcatT_hbm, xcf_hbm, tables_hbm, out_hbm, idx_v, fgath, cont_v, outbuf, sem):
    wid = lax.axis_index("s") * NC + lax.axis_index("c")

    def chunk_body(c, carry):
        b0 = wid * BPW + c * CHUNK
        pltpu.sync_copy(xcatT_hbm.at[:, pl.ds(b0, CHUNK)], idx_v)
        pltpu.sync_copy(
            xcf_hbm.at[pl.ds(b0 * CONT, CHUNK * CONT)],
            cont_v.at[pl.ds(0, CHUNK * CONT)],
        )
        copies = [
            pltpu.async_copy(
                tables_hbm.at[f].at[idx_v.at[f]],
                fgath.at[pl.ds(f * CHUNK, CHUNK)],
                sem,
            )
            for f in range(NUM_FIELDS)
        ]
        for cp in copies:
            cp.wait()

        def row_body(i, carry2):
            outbuf[i, pl.ds(0, 16)] = cont_v[pl.ds(i * CONT, 16)]
            for f in range(NUM_FIELDS):
                outbuf[i, pl.ds(CONT + f * EMB, EMB)] = fgath[f * CHUNK + i, :]
            return carry2

        lax.fori_loop(0, CHUNK, row_body, 0)
        pltpu.sync_copy(outbuf, out_hbm.at[pl.ds(b0, CHUNK)])
        return carry

    lax.fori_loop(0, NCHUNK, chunk_body, 0)


def kernel(x_cont, x_cat, tables):
    tq = tables.transpose(0, 2, 1)
    tail = tables[:, TAIL_V:, :].reshape(NUM_FIELDS, 4, 128)
    tr = _sc_format(tq, tail)
    xcatT = x_cat.T
    xcf = x_cont.reshape(-1)
    return _sc_embed(xcatT, xcf, tr.reshape(NUM_FIELDS, VPAD, EMB))


# speedup vs baseline: 2.6266x; 1.9694x over previous
import functools

import jax
import jax.numpy as jnp
from jax import lax
from jax.experimental import pallas as pl
from jax.experimental.pallas import tpu as pltpu
from jax.experimental.pallas import tpu_sc as plsc

NUM_FIELDS = 26
VOCAB = 100000
EMB = 16
BATCH = 16384
CONT = 13
OUT_W = CONT + NUM_FIELDS * EMB

VPAD = 100096
NTILE = 781
TAIL_V = NTILE * 128
TAIL_N = VOCAB - TAIL_V

_info = plsc.get_sparse_core_info()
NC, NS = _info.num_cores, _info.num_subcores
NW = NC * NS
BPW = BATCH // NW
CHUNK = 128
NCHUNK = BPW // CHUNK

_mesh = plsc.VectorSubcoreMesh(core_axis_name="c", subcore_axis_name="s")


@functools.partial(
    pl.kernel,
    out_type=jax.ShapeDtypeStruct(
        (NUM_FIELDS, VPAD * EMB // 1024, 8, 128), jnp.float32
    ),
    mesh=_mesh,
    compiler_params=pltpu.CompilerParams(
        use_tc_tiling_on_sc=True, needs_layout_passes=False
    ),
    scratch_types=[
        pltpu.VMEM((EMB, 24 * 128), jnp.float32),
        pltpu.VMEM((48, 8, 128), jnp.float32),
        pltpu.VMEM((EMB, 128), jnp.float32),
        pltpu.VMEM((2, 8, 128), jnp.float32),
        pltpu.VMEM((4, 128), jnp.float32),
        pltpu.SemaphoreType.DMA,
        pltpu.SemaphoreType.DMA,
    ],
)
def _sc_format(tq_hbm, tail_hbm, tr_hbm, tilebuf, rowbuf, tb1, rb1, tailbuf, isem, osem):
    wid = lax.axis_index("s") * NC + lax.axis_index("c")
    base = wid * 24
    iota = lax.iota(jnp.int32, 16)

    def in_copy(f):
        return pltpu.make_async_copy(
            tq_hbm.at[f, :, pl.ds(base * 128, 24 * 128)], tilebuf, isem
        )

    def out_copy(f):
        return pltpu.make_async_copy(
            rowbuf, tr_hbm.at[f, pl.ds(base * 2, 48)], osem
        )

    in_copy(0).start()
    iota_hi = iota >> 3
    iota_lo16 = (iota & 7) * 16

    def fbody(f, carry):
        in_copy(f).wait()

        @pl.when(f > 0)
        def _():
            out_copy(f - 1).wait()

        def tblock(t, carry2):
            for k in range(8):
                r3 = jnp.full((16,), 0, jnp.int32) + (2 * t + (1 if k >= 4 else 0))
                r8 = iota_hi + (2 * k) % 8
                for e in range(16):
                    seg = tilebuf[e, pl.ds(t * 128 + k * 16, 16)]
                    plsc.store_scatter(rowbuf, [r3, r8, iota_lo16 + e], seg)
            return carry2

        lax.fori_loop(0, 24, tblock, 0)
        out_copy(f).start()

        @pl.when(f < NUM_FIELDS - 1)
        def _():
            in_copy(f + 1).start()

        return carry

    lax.fori_loop(0, NUM_FIELDS, fbody, 0)
    out_copy(NUM_FIELDS - 1).wait()

    @pl.when(wid < 13)
    def _():
        j = 768 + wid

        def lbody(f, carry):
            pltpu.sync_copy(tq_hbm.at[f, :, pl.ds(j * 128, 128)], tb1)
            for k in range(8):
                r3 = jnp.full((16,), 0, jnp.int32) + (1 if k >= 4 else 0)
                r8 = iota_hi + (2 * k) % 8
                for e in range(16):
                    seg = tb1[e, pl.ds(k * 16, 16)]
                    plsc.store_scatter(rb1, [r3, r8, iota_lo16 + e], seg)
            pltpu.sync_copy(rb1, tr_hbm.at[f, pl.ds(j * 2, 2)])
            return carry

        lax.fori_loop(0, NUM_FIELDS, lbody, 0)

    @pl.when(wid < NUM_FIELDS)
    def _():
        pltpu.sync_copy(tail_hbm.at[wid], tailbuf)
        pltpu.sync_copy(
            tailbuf, tr_hbm.at[wid, TAIL_V * EMB // 1024, pl.ds(0, 4)]
        )


@functools.partial(
    pl.kernel,
    out_type=jax.ShapeDtypeStruct((BATCH, OUT_W), jnp.float32),
    mesh=_mesh,
    compiler_params=pltpu.CompilerParams(use_tc_tiling_on_sc=False),
    scratch_types=[
        pltpu.VMEM((NUM_FIELDS, CHUNK), jnp.int32),
        pltpu.VMEM((NUM_FIELDS * CHUNK, EMB), jnp.float32),
        pltpu.VMEM((CHUNK * CONT + 16,), jnp.float32),
        pltpu.VMEM((CHUNK, OUT_W), jnp.float32),
        pltpu.SemaphoreType.DMA,
    ],
)
def _sc_embed(xcatT_hbm, xcf_hbm, tables_hbm, out_hbm, idx_v, fgath, cont_v, outbuf, sem):
    wid = lax.axis_index("s") * NC + lax.axis_index("c")

    def chunk_body(c, carry):
        b0 = wid * BPW + c * CHUNK
        pltpu.sync_copy(xcatT_hbm.at[:, pl.ds(b0, CHUNK)], idx_v)
        pltpu.sync_copy(
            xcf_hbm.at[pl.ds(b0 * CONT, CHUNK * CONT)],
            cont_v.at[pl.ds(0, CHUNK * CONT)],
        )
        copies = [
            pltpu.async_copy(
                tables_hbm.at[f].at[idx_v.at[f]],
                fgath.at[pl.ds(f * CHUNK, CHUNK)],
                sem,
            )
            for f in range(NUM_FIELDS)
        ]
        for cp in copies:
            cp.wait()

        def row_body(i, carry2):
            outbuf[i, pl.ds(0, 16)] = cont_v[pl.ds(i * CONT, 16)]
            for f in range(NUM_FIELDS):
                outbuf[i, pl.ds(CONT + f * EMB, EMB)] = fgath[f * CHUNK + i, :]
            return carry2

        lax.fori_loop(0, CHUNK, row_body, 0)
        pltpu.sync_copy(outbuf, out_hbm.at[pl.ds(b0, CHUNK)])
        return carry

    lax.fori_loop(0, NCHUNK, chunk_body, 0)


def kernel(x_cont, x_cat, tables):
    tq = tables.transpose(0, 2, 1)
    tail = tables[:, TAIL_V:, :].reshape(NUM_FIELDS, 4, 128)
    tr = _sc_format(tq, tail)
    xcatT = x_cat.T
    xcf = x_cont.reshape(-1)
    return _sc_embed(xcatT, xcf, tr.reshape(NUM_FIELDS, VPAD, EMB))


# speedup vs baseline: 3.3059x; 1.2587x over previous
import functools

import jax
import jax.numpy as jnp
from jax import lax
from jax.experimental import pallas as pl
from jax.experimental.pallas import tpu as pltpu
from jax.experimental.pallas import tpu_sc as plsc

NUM_FIELDS = 26
VOCAB = 100000
EMB = 16
BATCH = 16384
CONT = 13
OUT_W = CONT + NUM_FIELDS * EMB

VPAD = 100096
NTILE = 781
TAIL_V = NTILE * 128
TAIL_N = VOCAB - TAIL_V

_info = plsc.get_sparse_core_info()
NC, NS = _info.num_cores, _info.num_subcores
NW = NC * NS
BPW = BATCH // NW
CHUNK = 128
NCHUNK = BPW // CHUNK

_mesh = plsc.VectorSubcoreMesh(core_axis_name="c", subcore_axis_name="s")


@functools.partial(
    pl.kernel,
    out_type=jax.ShapeDtypeStruct(
        (NUM_FIELDS, VPAD * EMB // 1024, 8, 128), jnp.float32
    ),
    mesh=_mesh,
    compiler_params=pltpu.CompilerParams(
        use_tc_tiling_on_sc=True, needs_layout_passes=False
    ),
    scratch_types=[
        pltpu.VMEM((EMB, 12 * 128), jnp.float32),
        pltpu.VMEM((EMB, 12 * 128), jnp.float32),
        pltpu.VMEM((24, 8, 128), jnp.float32),
        pltpu.VMEM((24, 8, 128), jnp.float32),
        pltpu.SemaphoreType.DMA,
        pltpu.SemaphoreType.DMA,
        pltpu.SemaphoreType.DMA,
        pltpu.SemaphoreType.DMA,
    ],
)
def _sc_format(
    tq_hbm, tail_hbm, tr_hbm,
    tileA, tileB, rowA, rowB, isemA, isemB, osemA, osemB,
):
    wid = lax.axis_index("s") * NC + lax.axis_index("c")
    base = wid * 24
    iota = lax.iota(jnp.int32, 16)

    def in_copy(f, h, tile, isem):
        return pltpu.make_async_copy(
            tq_hbm.at[f, :, pl.ds(base * 128 + h * 12 * 128, 12 * 128)], tile, isem
        )

    def out_copy(f, h, row, osem):
        return pltpu.make_async_copy(
            row, tr_hbm.at[f, pl.ds(base * 2 + h * 24, 24)], osem
        )

    in_copy(0, 0, tileA, isemA).start()
    iota_hi = iota >> 3
    iota_lo16 = (iota & 7) * 16

    def transpose_field(tile, row):
        def tblock(t, carry2):
            for k in range(8):
                r3 = jnp.full((16,), 0, jnp.int32) + (2 * t + (1 if k >= 4 else 0))
                r8 = iota_hi + (2 * k) % 8
                for e in range(16):
                    seg = tile[e, pl.ds(t * 128 + k * 16, 16)]
                    plsc.store_scatter(row, [r3, r8, iota_lo16 + e], seg)
            return carry2

        lax.fori_loop(0, 12, tblock, 0)

    def fbody(f, carry):
        in_copy(f, 0, tileA, isemA).wait()
        in_copy(f, 1, tileB, isemB).start()

        @pl.when(f > 0)
        def _():
            out_copy(f - 1, 0, rowA, osemA).wait()
            out_copy(f - 1, 1, rowB, osemB).wait()

        transpose_field(tileA, rowA)
        out_copy(f, 0, rowA, osemA).start()
        in_copy(f, 1, tileB, isemB).wait()

        @pl.when(f < NUM_FIELDS - 1)
        def _():
            in_copy(f + 1, 0, tileA, isemA).start()

        transpose_field(tileB, rowB)
        out_copy(f, 1, rowB, osemB).start()
        return carry

    lax.fori_loop(0, NUM_FIELDS, fbody, 0)
    out_copy(NUM_FIELDS - 1, 0, rowA, osemA).wait()
    out_copy(NUM_FIELDS - 1, 1, rowB, osemB).wait()

    @pl.when(wid < 13)
    def _():
        j = 768 + wid

        def lbody(f, carry):
            pltpu.sync_copy(tq_hbm.at[f, :, pl.ds(j * 128, 128)], tileA.at[:, pl.ds(0, 128)])
            for k in range(8):
                r3 = jnp.full((16,), 0, jnp.int32) + (1 if k >= 4 else 0)
                r8 = iota_hi + (2 * k) % 8
                for e in range(16):
                    seg = tileA[e, pl.ds(k * 16, 16)]
                    plsc.store_scatter(rowA, [r3, r8, iota_lo16 + e], seg)
            pltpu.sync_copy(rowA.at[pl.ds(0, 2)], tr_hbm.at[f, pl.ds(j * 2, 2)])
            return carry

        lax.fori_loop(0, NUM_FIELDS, lbody, 0)

    @pl.when(wid < NUM_FIELDS)
    def _():
        pltpu.sync_copy(tail_hbm.at[wid], rowA.at[0, pl.ds(0, 4)])
        pltpu.sync_copy(
            rowA.at[0, pl.ds(0, 4)],
            tr_hbm.at[wid, TAIL_V * EMB // 1024, pl.ds(0, 4)],
        )


@functools.partial(
    pl.kernel,
    out_type=jax.ShapeDtypeStruct((BATCH, OUT_W), jnp.float32),
    mesh=_mesh,
    compiler_params=pltpu.CompilerParams(use_tc_tiling_on_sc=False),
    scratch_types=[
        pltpu.VMEM((NUM_FIELDS, CHUNK), jnp.int32),
        pltpu.VMEM((NUM_FIELDS * CHUNK, EMB), jnp.float32),
        pltpu.VMEM((CHUNK * CONT + 16,), jnp.float32),
        pltpu.VMEM((CHUNK, OUT_W), jnp.float32),
        pltpu.SemaphoreType.DMA,
    ],
)
def _sc_embed(xcatT_hbm, xcf_hbm, tables_hbm, out_hbm, idx_v, fgath, cont_v, outbuf, sem):
    wid = lax.axis_index("s") * NC + lax.axis_index("c")

    def chunk_body(c, carry):
        b0 = wid * BPW + c * CHUNK
        pltpu.sync_copy(xcatT_hbm.at[:, pl.ds(b0, CHUNK)], idx_v)
        pltpu.sync_copy(
            xcf_hbm.at[pl.ds(b0 * CONT, CHUNK * CONT)],
            cont_v.at[pl.ds(0, CHUNK * CONT)],
        )
        copies = [
            pltpu.async_copy(
                tables_hbm.at[f].at[idx_v.at[f]],
                fgath.at[pl.ds(f * CHUNK, CHUNK)],
                sem,
            )
            for f in range(NUM_FIELDS)
        ]
        for cp in copies:
            cp.wait()

        def row_body(i, carry2):
            outbuf[i, pl.ds(0, 16)] = cont_v[pl.ds(i * CONT, 16)]
            for f in range(NUM_FIELDS):
                outbuf[i, pl.ds(CONT + f * EMB, EMB)] = fgath[f * CHUNK + i, :]
            return carry2

        lax.fori_loop(0, CHUNK, row_body, 0)
        pltpu.sync_copy(outbuf, out_hbm.at[pl.ds(b0, CHUNK)])
        return carry

    lax.fori_loop(0, NCHUNK, chunk_body, 0)


def kernel(x_cont, x_cat, tables):
    tq = tables.transpose(0, 2, 1)
    tail = tables[:, TAIL_V:, :].reshape(NUM_FIELDS, 4, 128)
    tr = _sc_format(tq, tail)
    xcatT = x_cat.T
    xcf = x_cont.reshape(-1)
    return _sc_embed(xcatT, xcf, tr.reshape(NUM_FIELDS, VPAD, EMB))
